# Initial kernel scaffold; baseline (speedup 1.0000x reference)
import jax
import jax.numpy as jnp
from jax.experimental import pallas as pl


def kernel(x):
    raise NotImplementedError("write your pallas kernel here")


# speedup vs baseline: 2.8583x; 2.8583x over previous
import jax
import jax.numpy as jnp
from jax.experimental import pallas as pl
from jax.experimental.pallas import tpu as pltpu

_BS = 512


def _scan_body(x_ref, o_ref, carry_ref):
    j = pl.program_id(1)

    @pl.when(j == 0)
    def _():
        carry_ref[...] = jnp.zeros_like(carry_ref)

    blk = x_ref[0]
    c = blk
    d = 1
    while d < _BS:
        shifted = jnp.concatenate(
            [jnp.zeros((d, c.shape[1]), c.dtype), c[: _BS - d]], axis=0
        )
        c = c + shifted
        d *= 2
    c = c + carry_ref[0][None, :]
    o_ref[0] = c
    carry_ref[0] = c[_BS - 1]


def kernel(x):
    B, S, L = x.shape
    grid = (B, S // _BS)
    return pl.pallas_call(
        _scan_body,
        grid=grid,
        in_specs=[pl.BlockSpec((1, _BS, L), lambda i, j: (i, j, 0))],
        out_specs=pl.BlockSpec((1, _BS, L), lambda i, j: (i, j, 0)),
        out_shape=jax.ShapeDtypeStruct(x.shape, x.dtype),
        scratch_shapes=[pltpu.VMEM((1, L), jnp.float32)],
        compiler_params=pltpu.CompilerParams(
            dimension_semantics=("arbitrary", "arbitrary"),
        ),
    )(x)


# speedup vs baseline: 3.0110x; 1.0534x over previous
import jax
import jax.numpy as jnp
from jax.experimental import pallas as pl
from jax.experimental.pallas import tpu as pltpu

_BS = 1024


def _scan_body(x_ref, o_ref, carry_ref):
    j = pl.program_id(1)

    @pl.when(j == 0)
    def _():
        carry_ref[...] = jnp.zeros_like(carry_ref)

    blk = x_ref[0]
    c = blk
    d = 1
    while d < _BS:
        shifted = jnp.concatenate(
            [jnp.zeros((d, c.shape[1]), c.dtype), c[: _BS - d]], axis=0
        )
        c = c + shifted
        d *= 2
    c = c + carry_ref[0][None, :]
    o_ref[0] = c
    carry_ref[0] = c[_BS - 1]


def kernel(x):
    B, S, L = x.shape
    grid = (B, S // _BS)
    return pl.pallas_call(
        _scan_body,
        grid=grid,
        in_specs=[pl.BlockSpec((1, _BS, L), lambda i, j: (i, j, 0))],
        out_specs=pl.BlockSpec((1, _BS, L), lambda i, j: (i, j, 0)),
        out_shape=jax.ShapeDtypeStruct(x.shape, x.dtype),
        scratch_shapes=[pltpu.VMEM((1, L), jnp.float32)],
        compiler_params=pltpu.CompilerParams(
            dimension_semantics=("arbitrary", "arbitrary"),
        ),
    )(x)


# speedup vs baseline: 3.3590x; 1.1156x over previous
import jax
import jax.numpy as jnp
from jax.experimental import pallas as pl
from jax.experimental.pallas import tpu as pltpu

_BS = 1024


def _scan_body(x_ref, o_ref, carry_ref):
    j = pl.program_id(1)

    @pl.when(j == 0)
    def _():
        carry_ref[...] = jnp.zeros_like(carry_ref)

    L = x_ref.shape[2]

    def group(k, carry):
        v = x_ref[0, pl.ds(k * 8, 8), :]
        for d in (1, 2, 4):
            v = v + jnp.concatenate(
                [jnp.zeros((d, L), v.dtype), v[: 8 - d]], axis=0
            )
        v = v + carry
        o_ref[0, pl.ds(k * 8, 8), :] = v
        return v[7:8, :]

    carry_ref[...] = jax.lax.fori_loop(0, _BS // 8, group, carry_ref[...])


def kernel(x):
    B, S, L = x.shape
    grid = (B, S // _BS)
    return pl.pallas_call(
        _scan_body,
        grid=grid,
        in_specs=[pl.BlockSpec((1, _BS, L), lambda i, j: (i, j, 0))],
        out_specs=pl.BlockSpec((1, _BS, L), lambda i, j: (i, j, 0)),
        out_shape=jax.ShapeDtypeStruct(x.shape, x.dtype),
        scratch_shapes=[pltpu.VMEM((1, L), jnp.float32)],
        compiler_params=pltpu.CompilerParams(
            dimension_semantics=("arbitrary", "arbitrary"),
        ),
    )(x)


# speedup vs baseline: 3.4661x; 1.0319x over previous
import jax
import jax.numpy as jnp
from jax.experimental import pallas as pl
from jax.experimental.pallas import tpu as pltpu

_BS = 1024
_U = 4


def _scan_body(x_ref, o_ref, carry_ref):
    j = pl.program_id(1)

    @pl.when(j == 0)
    def _():
        carry_ref[...] = jnp.zeros_like(carry_ref)

    L = x_ref.shape[2]

    def group(k, carry):
        vs = []
        for u in range(_U):
            v = x_ref[0, pl.ds(k * (8 * _U) + u * 8, 8), :]
            for d in (1, 2, 4):
                v = v + jnp.concatenate(
                    [jnp.zeros((d, L), v.dtype), v[: 8 - d]], axis=0
                )
            vs.append(v)
        offs = [carry]
        for u in range(_U - 1):
            offs.append(offs[-1] + vs[u][7:8, :])
        for u in range(_U):
            o_ref[0, pl.ds(k * (8 * _U) + u * 8, 8), :] = vs[u] + offs[u]
        return offs[_U - 1] + vs[_U - 1][7:8, :]

    carry_ref[...] = jax.lax.fori_loop(0, _BS // (8 * _U), group, carry_ref[...])


def kernel(x):
    B, S, L = x.shape
    grid = (B, S // _BS)
    return pl.pallas_call(
        _scan_body,
        grid=grid,
        in_specs=[pl.BlockSpec((1, _BS, L), lambda i, j: (i, j, 0))],
        out_specs=pl.BlockSpec((1, _BS, L), lambda i, j: (i, j, 0)),
        out_shape=jax.ShapeDtypeStruct(x.shape, x.dtype),
        scratch_shapes=[pltpu.VMEM((1, L), jnp.float32)],
        compiler_params=pltpu.CompilerParams(
            dimension_semantics=("arbitrary", "arbitrary"),
        ),
    )(x)
